# baseline (speedup 1.0000x reference)
import jax
import jax.numpy as jnp
from jax import lax
from jax.experimental import pallas as pl
from jax.experimental.pallas import tpu as pltpu
from jax.experimental.pallas import tpu_sc as plsc

F32 = jnp.float32
I32 = jnp.int32

NC = 2
NS = 16
L = 16
NW = NC * NS
SUB = 128

_SC_PARAMS = dict(
    compiler_params=pltpu.CompilerParams(
        needs_layout_passes=False, use_tc_tiling_on_sc=False),
)


def _sc_mesh():
    return plsc.VectorSubcoreMesh(core_axis_name="c", subcore_axis_name="s")


def _make_pass1(E, n, n_pad):
    CHUNK = 512
    NSUB = CHUNK // SUB
    GP = CHUNK // L
    n_chunks = E // CHUNK
    max_chunks = -(-n_chunks // NW)
    zrows = n_pad // NS

    def body(rows_h, cols_h, attr32_h,
             sa_h, sl_h, cr_h,
             r_v, c_v, a32_v, ones_v, ia_v, ib_v, ic_v, z32_v, z16_v,
             sa_s, sl_s, cr_s):
        cid = lax.axis_index("c")
        sid = lax.axis_index("s")
        wid = cid * NS + sid
        zero16 = jnp.zeros((L,), F32)
        ones16 = jnp.ones((L,), F32)
        trash16 = jnp.full((L,), n, I32)

        def _z0(i, carry):
            z32_v[i, pl.ds(0, L)] = zero16
            z32_v[i, pl.ds(L, L)] = zero16
            z16_v[i, :] = zero16
            return carry
        lax.fori_loop(0, 64, _z0, 0)

        def _o(i, carry):
            ones_v[i, :] = ones16
            return carry
        lax.fori_loop(0, SUB, _o, 0)

        def _zs(i, carry):
            r0 = sid * zrows + i * 64
            pltpu.sync_copy(z32_v, sa_s.at[pl.ds(r0, 64)])
            pltpu.sync_copy(z32_v, sl_s.at[pl.ds(r0, 64)])
            pltpu.sync_copy(z16_v, cr_s.at[pl.ds(r0, 64)])
            return carry
        lax.fori_loop(0, zrows // 64, _zs, 0)
        plsc.subcore_barrier()

        def chunk_body(k, carry):
            i = wid + k * NW

            @pl.when(i < n_chunks)
            def _():
                base = i * CHUNK
                pltpu.sync_copy(rows_h.at[pl.ds(base, CHUNK)], r_v)
                pltpu.sync_copy(cols_h.at[pl.ds(base, CHUNK)], c_v)
                pltpu.sync_copy(attr32_h.at[pl.ds(base, CHUNK)], a32_v)

                def grp(g, carry2):
                    r16 = r_v[pl.ds(g * L, L)]
                    c16 = c_v[pl.ds(g * L, L)]
                    m = r16 != c16
                    j = g // (SUB // L)
                    o = (g % (SUB // L)) * L
                    ia_v[j, pl.ds(o, L)] = c16
                    ib_v[j, pl.ds(o, L)] = jnp.where(m, trash16, c16)
                    ic_v[j, pl.ds(o, L)] = jnp.where(m, r16, trash16)
                    return carry2
                lax.fori_loop(0, GP, grp, 0)
                for j in range(NSUB):
                    pltpu.sync_copy(a32_v.at[pl.ds(j * SUB, SUB)],
                                    sa_s.at[ia_v.at[j]], add=True)
                    pltpu.sync_copy(a32_v.at[pl.ds(j * SUB, SUB)],
                                    sl_s.at[ib_v.at[j]], add=True)
                    pltpu.sync_copy(ones_v, cr_s.at[ic_v.at[j]], add=True)
            return carry
        lax.fori_loop(0, max_chunks, chunk_body, 0)
        plsc.subcore_barrier()

        r0 = sid * zrows
        pltpu.sync_copy(sa_s.at[pl.ds(r0, zrows)], sa_h.at[cid, pl.ds(r0, zrows)])
        pltpu.sync_copy(sl_s.at[pl.ds(r0, zrows)], sl_h.at[cid, pl.ds(r0, zrows)])
        pltpu.sync_copy(cr_s.at[pl.ds(r0, zrows)], cr_h.at[cid, pl.ds(r0, zrows)])

    return pl.kernel(
        body,
        out_type=(jax.ShapeDtypeStruct((NC, n_pad, 32), F32),
                  jax.ShapeDtypeStruct((NC, n_pad, 32), F32),
                  jax.ShapeDtypeStruct((NC, n_pad, 16), F32)),
        mesh=_sc_mesh(),
        scratch_types=[
            pltpu.VMEM((CHUNK,), I32),
            pltpu.VMEM((CHUNK,), I32),
            pltpu.VMEM((CHUNK, 32), F32),
            pltpu.VMEM((SUB, 16), F32),
            pltpu.VMEM((NSUB, SUB), I32),
            pltpu.VMEM((NSUB, SUB), I32),
            pltpu.VMEM((NSUB, SUB), I32),
            pltpu.VMEM((64, 32), F32),
            pltpu.VMEM((64, 16), F32),
            pltpu.VMEM_SHARED((n_pad, 32), F32),
            pltpu.VMEM_SHARED((n_pad, 32), F32),
            pltpu.VMEM_SHARED((n_pad, 16), F32),
        ],
        **_SC_PARAMS,
    )


def _make_pass2a(E, n, n_pad):
    CHUNK = 256
    NSUB = CHUNK // SUB
    GP = CHUNK // L
    n_chunks = E // CHUNK
    max_chunks = -(-n_chunks // NW)
    zrows = n_pad // NS

    def body(rows_h, cols_h, y_h,
             acc_h,
             r_v, c_v, ir_v, ic_v, y_v, z_v,
             acc_s, sem):
        cid = lax.axis_index("c")
        sid = lax.axis_index("s")
        wid = cid * NS + sid
        zero16 = jnp.zeros((L,), F32)
        trash16 = jnp.full((L,), n, I32)

        def _zz(i, carry):
            for q in range(128 // L):
                z_v[i, pl.ds(q * L, L)] = zero16
            return carry
        lax.fori_loop(0, 32, _zz, 0)

        def _zs(i, carry):
            pltpu.sync_copy(z_v, acc_s.at[pl.ds(sid * zrows + i * 32, 32)])
            return carry
        lax.fori_loop(0, zrows // 32, _zs, 0)
        plsc.subcore_barrier()

        def chunk_body(k, carry):
            i = wid + k * NW

            @pl.when(i < n_chunks)
            def _():
                base = i * CHUNK
                pltpu.sync_copy(rows_h.at[pl.ds(base, CHUNK)], r_v)
                pltpu.sync_copy(cols_h.at[pl.ds(base, CHUNK)], c_v)

                def grp(g, carry2):
                    r16 = r_v[pl.ds(g * L, L)]
                    c16 = c_v[pl.ds(g * L, L)]
                    csc = jnp.where(r16 != c16, c16, trash16)
                    j = g // (SUB // L)
                    o = (g % (SUB // L)) * L
                    ir_v[j, pl.ds(o, L)] = r16
                    ic_v[j, pl.ds(o, L)] = csc
                    return carry2
                lax.fori_loop(0, GP, grp, 0)

                descs = []
                for j in range(NSUB):
                    descs.append(pltpu.async_copy(
                        y_h.at[ir_v.at[j]],
                        y_v.at[pl.ds(j * SUB, SUB)], sem))
                for d in descs:
                    d.wait()
                for j in range(NSUB):
                    pltpu.sync_copy(y_v.at[pl.ds(j * SUB, SUB)],
                                    acc_s.at[ic_v.at[j]], add=True)
            return carry
        lax.fori_loop(0, max_chunks, chunk_body, 0)
        plsc.subcore_barrier()

        r0 = sid * zrows
        pltpu.sync_copy(acc_s.at[pl.ds(r0, zrows)],
                        acc_h.at[cid, pl.ds(r0, zrows)])

    return pl.kernel(
        body,
        out_type=jax.ShapeDtypeStruct((NC, n_pad, 128), F32),
        mesh=_sc_mesh(),
        scratch_types=[
            pltpu.VMEM((CHUNK,), I32),
            pltpu.VMEM((CHUNK,), I32),
            pltpu.VMEM((NSUB, SUB), I32),
            pltpu.VMEM((NSUB, SUB), I32),
            pltpu.VMEM((CHUNK, 128), F32),
            pltpu.VMEM((32, 128), F32),
            pltpu.VMEM_SHARED((n_pad, 128), F32),
            pltpu.SemaphoreType.DMA,
        ],
        **_SC_PARAMS,
    )


def _make_pass2b(E, n, n_pad):
    CHUNK = 512
    NSUB = CHUNK // SUB
    GP = CHUNK // L
    n_chunks = E // CHUNK
    max_chunks = -(-n_chunks // NW)
    zrows = n_pad // NS

    def body(rows_h, cols_h, attr_h, t32_h,
             acc_h,
             r_v, c_v, a_v, ir_v, ic_v, t_v, z_v,
             acc_s, sem):
        cid = lax.axis_index("c")
        sid = lax.axis_index("s")
        wid = cid * NS + sid
        zero16 = jnp.zeros((L,), F32)
        trash16 = jnp.full((L,), n, I32)

        def _zz(i, carry):
            z_v[i, pl.ds(0, L)] = zero16
            z_v[i, pl.ds(L, L)] = zero16
            return carry
        lax.fori_loop(0, 64, _zz, 0)

        def _zs(i, carry):
            pltpu.sync_copy(z_v, acc_s.at[pl.ds(sid * zrows + i * 64, 64)])
            return carry
        lax.fori_loop(0, zrows // 64, _zs, 0)
        plsc.subcore_barrier()

        def chunk_body(k, carry):
            i = wid + k * NW

            @pl.when(i < n_chunks)
            def _():
                base = i * CHUNK
                pltpu.sync_copy(rows_h.at[pl.ds(base, CHUNK)], r_v)
                pltpu.sync_copy(cols_h.at[pl.ds(base, CHUNK)], c_v)
                pltpu.sync_copy(attr_h.at[pl.ds(base, CHUNK)], a_v)

                def grp(g, carry2):
                    r16 = r_v[pl.ds(g * L, L)]
                    c16 = c_v[pl.ds(g * L, L)]
                    csc = jnp.where(r16 != c16, c16, trash16)
                    j = g // (SUB // L)
                    o = (g % (SUB // L)) * L
                    ir_v[j, pl.ds(o, L)] = r16
                    ic_v[j, pl.ds(o, L)] = csc
                    return carry2
                lax.fori_loop(0, GP, grp, 0)

                descs = []
                for j in range(NSUB):
                    descs.append(pltpu.async_copy(
                        t32_h.at[ir_v.at[j]],
                        t_v.at[pl.ds(j * SUB, SUB)], sem))
                for d in descs:
                    d.wait()

                def edge(e, carry2):
                    t_v[e, pl.ds(0, L)] = t_v[e, pl.ds(0, L)] * a_v[e, :]
                    return carry2
                lax.fori_loop(0, CHUNK, edge, 0)

                for j in range(NSUB):
                    pltpu.sync_copy(t_v.at[pl.ds(j * SUB, SUB)],
                                    acc_s.at[ic_v.at[j]], add=True)
            return carry
        lax.fori_loop(0, max_chunks, chunk_body, 0)
        plsc.subcore_barrier()

        r0 = sid * zrows
        pltpu.sync_copy(acc_s.at[pl.ds(r0, zrows)],
                        acc_h.at[cid, pl.ds(r0, zrows)])

    return pl.kernel(
        body,
        out_type=jax.ShapeDtypeStruct((NC, n_pad, 32), F32),
        mesh=_sc_mesh(),
        scratch_types=[
            pltpu.VMEM((CHUNK,), I32),
            pltpu.VMEM((CHUNK,), I32),
            pltpu.VMEM((CHUNK, 16), F32),
            pltpu.VMEM((NSUB, SUB), I32),
            pltpu.VMEM((NSUB, SUB), I32),
            pltpu.VMEM((CHUNK, 32), F32),
            pltpu.VMEM((64, 32), F32),
            pltpu.VMEM_SHARED((n_pad, 32), F32),
            pltpu.SemaphoreType.DMA,
        ],
        **_SC_PARAMS,
    )


def _make_tcmid(n_pad):
    BLK = 512
    grid = (n_pad // BLK,)

    def body(sa_ref, sl_ref, cr_ref, x_ref, y_ref, t32_ref, aux_ref):
        sa32 = sa_ref[0] + sa_ref[1]
        sl32 = sl_ref[0] + sl_ref[1]
        cr = cr_ref[0] + cr_ref[1]
        sa = sa32[:, :16]
        cnt_all = sa32[:, 16:17]
        sl = sl32[:, :16]
        slc = sl32[:, 16:17]
        cra = cr[:, 0:1]
        la_mean = sa / jnp.maximum(cnt_all, 1.0)
        la_sl = sl / jnp.maximum(slc, 1.0)
        flag = jnp.minimum(slc, 1.0)
        la = flag * la_sl + (1.0 - flag) * la_mean
        degw = sa - sl + la
        dis = jnp.where(degw > 0.0, lax.rsqrt(jnp.maximum(degw, 1e-30)), 0.0)
        dis2 = lax.rsqrt(cra + 1.0)
        y_ref[...] = dis2 * x_ref[...]
        t32_ref[:, :16] = dis2 * dis
        t32_ref[:, 16:] = jnp.concatenate(
            [dis2, jnp.zeros((BLK, 15), F32)], axis=1)
        aux_ref[:, :16] = la
        aux_ref[:, 16:] = jnp.broadcast_to(dis2, (BLK, 16))

    return pl.pallas_call(
        body,
        grid=grid,
        in_specs=[
            pl.BlockSpec((NC, BLK, 32), lambda i: (0, i, 0)),
            pl.BlockSpec((NC, BLK, 32), lambda i: (0, i, 0)),
            pl.BlockSpec((NC, BLK, 16), lambda i: (0, i, 0)),
            pl.BlockSpec((BLK, 128), lambda i: (i, 0)),
        ],
        out_specs=[
            pl.BlockSpec((BLK, 128), lambda i: (i, 0)),
            pl.BlockSpec((BLK, 32), lambda i: (i, 0)),
            pl.BlockSpec((BLK, 32), lambda i: (i, 0)),
        ],
        out_shape=(jax.ShapeDtypeStruct((n_pad, 128), F32),
                   jax.ShapeDtypeStruct((n_pad, 32), F32),
                   jax.ShapeDtypeStruct((n_pad, 32), F32)),
    )


def _make_tcpost(n_pad):
    BLK = 512
    grid = (n_pad // BLK,)

    def body(acc_ref, acc32_ref, y_ref, t32_ref, aux_ref,
             wet_ref, be_ref, wt_ref, bias_ref, out_ref):
        u = acc_ref[0] + acc_ref[1]
        v = acc32_ref[0] + acc32_ref[1]
        v16 = v[:, :16]
        vs_c = v[:, 16:17]
        y = y_ref[...]
        t = t32_ref[:, :16]
        la = aux_ref[:, :16]
        dis2 = aux_ref[:, 16:17]
        a = dis2 * (u + y)
        b = t * (v16 + t * la)
        s = dis2 * (vs_c + dis2)
        pre = (a + jnp.dot(b, wet_ref[...], preferred_element_type=F32)
               + s * be_ref[...])
        out_ref[...] = (jnp.dot(pre, wt_ref[...], preferred_element_type=F32)
                        + bias_ref[...])

    return pl.pallas_call(
        body,
        grid=grid,
        in_specs=[
            pl.BlockSpec((NC, BLK, 128), lambda i: (0, i, 0)),
            pl.BlockSpec((NC, BLK, 32), lambda i: (0, i, 0)),
            pl.BlockSpec((BLK, 128), lambda i: (i, 0)),
            pl.BlockSpec((BLK, 32), lambda i: (i, 0)),
            pl.BlockSpec((BLK, 32), lambda i: (i, 0)),
            pl.BlockSpec((16, 128), lambda i: (0, 0)),
            pl.BlockSpec((1, 128), lambda i: (0, 0)),
            pl.BlockSpec((128, 128), lambda i: (0, 0)),
            pl.BlockSpec((1, 128), lambda i: (0, 0)),
        ],
        out_specs=pl.BlockSpec((BLK, 128), lambda i: (i, 0)),
        out_shape=jax.ShapeDtypeStruct((n_pad, 128), F32),
    )


def kernel(x, edge_index, edge_attr, W, We, be, bias):
    n, d_in = x.shape
    E = edge_index.shape[1]
    n_pad = -(-(n + 1) // 1024) * 1024

    rows = edge_index[0].astype(I32)
    cols = edge_index[1].astype(I32)
    attr = edge_attr.astype(F32)
    attr32 = jnp.pad(attr, ((0, 0), (0, 16)), constant_values=1.0)
    x_pad = jnp.pad(x.astype(F32), ((0, n_pad - n), (0, 0)))

    sa32, sl32, cr16 = _make_pass1(E, n, n_pad)(rows, cols, attr32)
    y, t32, aux = _make_tcmid(n_pad)(sa32, sl32, cr16, x_pad)
    acc128 = _make_pass2a(E, n, n_pad)(rows, cols, y)
    acc32 = _make_pass2b(E, n, n_pad)(rows, cols, attr, t32)
    out = _make_tcpost(n_pad)(acc128, acc32, y, t32, aux,
                              We.T.astype(F32), be.astype(F32)[None],
                              W.T.astype(F32), bias.astype(F32)[None])
    return out[:n]


# speedup vs baseline: 1.3525x; 1.3525x over previous
import jax
import jax.numpy as jnp
from jax import lax
from jax.experimental import pallas as pl
from jax.experimental.pallas import tpu as pltpu
from jax.experimental.pallas import tpu_sc as plsc

F32 = jnp.float32
I32 = jnp.int32

NC = 2
NS = 16
L = 16
NW = NC * NS
SUB = 128

_SC_PARAMS = dict(
    compiler_params=pltpu.CompilerParams(
        needs_layout_passes=False, use_tc_tiling_on_sc=False),
)


def _sc_mesh():
    return plsc.VectorSubcoreMesh(core_axis_name="c", subcore_axis_name="s")


def _make_pass1(E, n, n_pad):
    CHUNK = 512
    NSUB = CHUNK // SUB
    GP = CHUNK // L
    n_chunks = E // CHUNK
    max_chunks = -(-n_chunks // NW)
    zrows = n_pad // NS

    def body(rows_h, cols_h, attr_h,
             sa_h, sl_h, cnt_h,
             r_v, c_v, a_v, ia_v, ib_v,
             cnt_v, slc_v, cra_v, z_v,
             sa_s, sl_s):
        cid = lax.axis_index("c")
        sid = lax.axis_index("s")
        wid = cid * NS + sid
        zero16 = jnp.zeros((L,), F32)
        ones16 = jnp.ones((L,), F32)
        trash16 = jnp.full((L,), n, I32)

        def _zz(i, carry):
            z_v[i, :] = zero16
            return carry
        lax.fori_loop(0, 64, _zz, 0)

        def _zc(i, carry):
            cnt_v[pl.ds(i * L, L)] = zero16
            slc_v[pl.ds(i * L, L)] = zero16
            cra_v[pl.ds(i * L, L)] = zero16
            return carry
        lax.fori_loop(0, n_pad // L, _zc, 0)

        def _zs(i, carry):
            r0 = sid * zrows + i * 64
            pltpu.sync_copy(z_v, sa_s.at[pl.ds(r0, 64)])
            pltpu.sync_copy(z_v, sl_s.at[pl.ds(r0, 64)])
            return carry
        lax.fori_loop(0, zrows // 64, _zs, 0)
        plsc.subcore_barrier()

        def chunk_body(k, carry):
            i = wid + k * NW

            @pl.when(i < n_chunks)
            def _():
                base = i * CHUNK
                pltpu.sync_copy(rows_h.at[pl.ds(base, CHUNK)], r_v)
                pltpu.sync_copy(cols_h.at[pl.ds(base, CHUNK)], c_v)
                pltpu.sync_copy(attr_h.at[pl.ds(base, CHUNK)], a_v)

                def grp(g, carry2):
                    r16 = r_v[pl.ds(g * L, L)]
                    c16 = c_v[pl.ds(g * L, L)]
                    m = r16 != c16
                    mf = jnp.where(m, 1.0, 0.0).astype(F32)
                    nmf = (1.0 - mf).astype(F32)
                    cb = jnp.where(m, trash16, c16)
                    j = g // (SUB // L)
                    o = (g % (SUB // L)) * L
                    ia_v[j, pl.ds(o, L)] = c16
                    ib_v[j, pl.ds(o, L)] = cb
                    plsc.addupdate_scatter(cnt_v, [c16], ones16)
                    plsc.addupdate_scatter(slc_v, [c16], nmf)
                    plsc.addupdate_scatter(cra_v, [r16], mf)
                    return carry2
                lax.fori_loop(0, GP, grp, 0)
                for j in range(NSUB):
                    pltpu.sync_copy(a_v.at[pl.ds(j * SUB, SUB)],
                                    sa_s.at[ia_v.at[j]], add=True)
                    pltpu.sync_copy(a_v.at[pl.ds(j * SUB, SUB)],
                                    sl_s.at[ib_v.at[j]], add=True)
            return carry
        lax.fori_loop(0, max_chunks, chunk_body, 0)
        plsc.subcore_barrier()

        r0 = sid * zrows
        pltpu.sync_copy(sa_s.at[pl.ds(r0, zrows)], sa_h.at[cid, pl.ds(r0, zrows)])
        pltpu.sync_copy(sl_s.at[pl.ds(r0, zrows)], sl_h.at[cid, pl.ds(r0, zrows)])
        pltpu.sync_copy(cnt_v, cnt_h.at[0, wid])
        pltpu.sync_copy(slc_v, cnt_h.at[1, wid])
        pltpu.sync_copy(cra_v, cnt_h.at[2, wid])

    return pl.kernel(
        body,
        out_type=(jax.ShapeDtypeStruct((NC, n_pad, 16), F32),
                  jax.ShapeDtypeStruct((NC, n_pad, 16), F32),
                  jax.ShapeDtypeStruct((3, NW, n_pad), F32)),
        mesh=_sc_mesh(),
        scratch_types=[
            pltpu.VMEM((CHUNK,), I32),
            pltpu.VMEM((CHUNK,), I32),
            pltpu.VMEM((CHUNK, 16), F32),
            pltpu.VMEM((NSUB, SUB), I32),
            pltpu.VMEM((NSUB, SUB), I32),
            pltpu.VMEM((n_pad,), F32),
            pltpu.VMEM((n_pad,), F32),
            pltpu.VMEM((n_pad,), F32),
            pltpu.VMEM((64, 16), F32),
            pltpu.VMEM_SHARED((n_pad, 16), F32),
            pltpu.VMEM_SHARED((n_pad, 16), F32),
        ],
        **_SC_PARAMS,
    )


def _make_pass2a(E, n, n_pad):
    CHUNK = SUB
    GP = CHUNK // L
    n_chunks = E // CHUNK
    max_chunks = -(-n_chunks // NW)
    if max_chunks % 2:
        max_chunks += 1
    zrows = n_pad // NS

    def body(ei_h, y_h,
             acc_h,
             i0_v, i1_v, ir0_v, ir1_v, ic0_v, ic1_v, y0_v, y1_v, z_v,
             acc_s, sem0, sem1):
        cid = lax.axis_index("c")
        sid = lax.axis_index("s")
        wid = cid * NS + sid
        zero16 = jnp.zeros((L,), F32)
        trash16 = jnp.full((L,), n, I32)
        bufs = ((i0_v, ir0_v, ic0_v, y0_v, sem0),
                (i1_v, ir1_v, ic1_v, y1_v, sem1))

        def _zz(i, carry):
            for q in range(128 // L):
                z_v[i, pl.ds(q * L, L)] = zero16
            return carry
        lax.fori_loop(0, 32, _zz, 0)

        def _zs(i, carry):
            pltpu.sync_copy(z_v, acc_s.at[pl.ds(sid * zrows + i * 32, 32)])
            return carry
        lax.fori_loop(0, zrows // 32, _zs, 0)
        plsc.subcore_barrier()

        def fire(k, buf):
            i_v, ir_v, ic_v, y_v, sem = buf
            i = wid + k * NW

            @pl.when(i < n_chunks)
            def _():
                base = i * CHUNK
                pltpu.sync_copy(ei_h.at[:, pl.ds(base, CHUNK)], i_v)

                def grp(g, carry2):
                    r16 = i_v[0, pl.ds(g * L, L)]
                    c16 = i_v[1, pl.ds(g * L, L)]
                    csc = jnp.where(r16 != c16, c16, trash16)
                    ir_v[0, pl.ds(g * L, L)] = r16
                    ic_v[0, pl.ds(g * L, L)] = csc
                    return carry2
                lax.fori_loop(0, GP, grp, 0)
                pltpu.async_copy(y_h.at[ir_v.at[0]], y_v, sem)

        def drain_scatter(k, buf):
            i_v, ir_v, ic_v, y_v, sem = buf
            i = wid + k * NW

            @pl.when(i < n_chunks)
            def _():
                pltpu.make_async_copy(y_h.at[ir_v.at[0]], y_v, sem).wait()
                pltpu.sync_copy(y_v, acc_s.at[ic_v.at[0]], add=True)

        fire(0, bufs[0])

        def chunk_body(k2, carry):
            k = 2 * k2
            fire(k + 1, bufs[1])
            drain_scatter(k, bufs[0])
            fire(k + 2, bufs[0])
            drain_scatter(k + 1, bufs[1])
            return carry
        lax.fori_loop(0, max_chunks // 2, chunk_body, 0)
        plsc.subcore_barrier()

        r0 = sid * zrows
        pltpu.sync_copy(acc_s.at[pl.ds(r0, zrows)],
                        acc_h.at[cid, pl.ds(r0, zrows)])

    return pl.kernel(
        body,
        out_type=jax.ShapeDtypeStruct((NC, n_pad, 128), F32),
        mesh=_sc_mesh(),
        scratch_types=[
            pltpu.VMEM((2, CHUNK), I32),
            pltpu.VMEM((2, CHUNK), I32),
            pltpu.VMEM((1, SUB), I32),
            pltpu.VMEM((1, SUB), I32),
            pltpu.VMEM((1, SUB), I32),
            pltpu.VMEM((1, SUB), I32),
            pltpu.VMEM((CHUNK, 128), F32),
            pltpu.VMEM((CHUNK, 128), F32),
            pltpu.VMEM((32, 128), F32),
            pltpu.VMEM_SHARED((n_pad, 128), F32),
            pltpu.SemaphoreType.DMA,
            pltpu.SemaphoreType.DMA,
        ],
        **_SC_PARAMS,
    )


def _make_pass2b(E, n, n_pad):
    CHUNK = 512
    NSUB = CHUNK // SUB
    GP = CHUNK // L
    n_chunks = E // CHUNK
    max_chunks = -(-n_chunks // NW)
    zrows = n_pad // NS

    def body(rows_h, cols_h, attr_h, t_h, dis2_h,
             acc_h, vs_h,
             r_v, c_v, a_v, ir_v, ic_v, t_v, d2_v, vs_v, z_v,
             acc_s, sem):
        cid = lax.axis_index("c")
        sid = lax.axis_index("s")
        wid = cid * NS + sid
        zero16 = jnp.zeros((L,), F32)
        trash16 = jnp.full((L,), n, I32)

        pltpu.sync_copy(dis2_h, d2_v)

        def _zz(i, carry):
            z_v[i, :] = zero16
            return carry
        lax.fori_loop(0, 64, _zz, 0)

        def _zc(i, carry):
            vs_v[pl.ds(i * L, L)] = zero16
            return carry
        lax.fori_loop(0, n_pad // L, _zc, 0)

        def _zs(i, carry):
            pltpu.sync_copy(z_v, acc_s.at[pl.ds(sid * zrows + i * 64, 64)])
            return carry
        lax.fori_loop(0, zrows // 64, _zs, 0)
        plsc.subcore_barrier()

        def chunk_body(k, carry):
            i = wid + k * NW

            @pl.when(i < n_chunks)
            def _():
                base = i * CHUNK
                pltpu.sync_copy(rows_h.at[pl.ds(base, CHUNK)], r_v)
                pltpu.sync_copy(cols_h.at[pl.ds(base, CHUNK)], c_v)
                pltpu.sync_copy(attr_h.at[pl.ds(base, CHUNK)], a_v)

                def grp(g, carry2):
                    r16 = r_v[pl.ds(g * L, L)]
                    c16 = c_v[pl.ds(g * L, L)]
                    csc = jnp.where(r16 != c16, c16, trash16)
                    j = g // (SUB // L)
                    o = (g % (SUB // L)) * L
                    ir_v[j, pl.ds(o, L)] = r16
                    ic_v[j, pl.ds(o, L)] = csc
                    d2r = plsc.load_gather(d2_v, [r16])
                    plsc.addupdate_scatter(vs_v, [csc], d2r)
                    return carry2
                lax.fori_loop(0, GP, grp, 0)

                descs = []
                for j in range(NSUB):
                    descs.append(pltpu.async_copy(
                        t_h.at[ir_v.at[j]],
                        t_v.at[pl.ds(j * SUB, SUB)], sem))
                for d in descs:
                    d.wait()

                def grp_mul(g, carry2):
                    for q in range(L):
                        e = g * L + q
                        t_v[e, :] = t_v[e, :] * a_v[e, :]
                    return carry2
                lax.fori_loop(0, GP, grp_mul, 0)

                for j in range(NSUB):
                    pltpu.sync_copy(t_v.at[pl.ds(j * SUB, SUB)],
                                    acc_s.at[ic_v.at[j]], add=True)
            return carry
        lax.fori_loop(0, max_chunks, chunk_body, 0)
        plsc.subcore_barrier()

        r0 = sid * zrows
        pltpu.sync_copy(acc_s.at[pl.ds(r0, zrows)],
                        acc_h.at[cid, pl.ds(r0, zrows)])
        pltpu.sync_copy(vs_v, vs_h.at[wid])

    return pl.kernel(
        body,
        out_type=(jax.ShapeDtypeStruct((NC, n_pad, 16), F32),
                  jax.ShapeDtypeStruct((NW, n_pad), F32)),
        mesh=_sc_mesh(),
        scratch_types=[
            pltpu.VMEM((CHUNK,), I32),
            pltpu.VMEM((CHUNK,), I32),
            pltpu.VMEM((CHUNK, 16), F32),
            pltpu.VMEM((NSUB, SUB), I32),
            pltpu.VMEM((NSUB, SUB), I32),
            pltpu.VMEM((CHUNK, 16), F32),
            pltpu.VMEM((n_pad,), F32),
            pltpu.VMEM((n_pad,), F32),
            pltpu.VMEM((64, 16), F32),
            pltpu.VMEM_SHARED((n_pad, 16), F32),
            pltpu.SemaphoreType.DMA,
        ],
        **_SC_PARAMS,
    )


def _make_tcmid(n_pad):
    BLK = 512
    grid = (n_pad // BLK,)

    def body(sa_ref, sl_ref, cnt_ref, x_ref, y_ref, t_ref, dis2_ref, aux_ref):
        sa = sa_ref[0] + sa_ref[1]
        sl = sl_ref[0] + sl_ref[1]
        cnt = jnp.sum(cnt_ref[...], axis=1)
        cnt_t = jnp.transpose(cnt, (1, 0))
        cnt_all = cnt_t[:, 0:1]
        slc = cnt_t[:, 1:2]
        cra = cnt_t[:, 2:3]
        la_mean = sa / jnp.maximum(cnt_all, 1.0)
        la_sl = sl / jnp.maximum(slc, 1.0)
        flag = jnp.minimum(slc, 1.0)
        la = flag * la_sl + (1.0 - flag) * la_mean
        degw = sa - sl + la
        dis = jnp.where(degw > 0.0, lax.rsqrt(jnp.maximum(degw, 1e-30)), 0.0)
        dis2 = lax.rsqrt(cra + 1.0)
        y_ref[...] = dis2 * x_ref[...]
        t_ref[...] = dis2 * dis
        dis2_ref[...] = lax.rsqrt(cnt[2:3, :] + 1.0)
        aux_ref[:, :16] = la
        aux_ref[:, 16:] = jnp.broadcast_to(dis2, (BLK, 16))

    return pl.pallas_call(
        body,
        grid=grid,
        in_specs=[
            pl.BlockSpec((NC, BLK, 16), lambda i: (0, i, 0)),
            pl.BlockSpec((NC, BLK, 16), lambda i: (0, i, 0)),
            pl.BlockSpec((3, NW, BLK), lambda i: (0, 0, i)),
            pl.BlockSpec((BLK, 128), lambda i: (i, 0)),
        ],
        out_specs=[
            pl.BlockSpec((BLK, 128), lambda i: (i, 0)),
            pl.BlockSpec((BLK, 16), lambda i: (i, 0)),
            pl.BlockSpec((1, BLK), lambda i: (0, i)),
            pl.BlockSpec((BLK, 32), lambda i: (i, 0)),
        ],
        out_shape=(jax.ShapeDtypeStruct((n_pad, 128), F32),
                   jax.ShapeDtypeStruct((n_pad, 16), F32),
                   jax.ShapeDtypeStruct((1, n_pad), F32),
                   jax.ShapeDtypeStruct((n_pad, 32), F32)),
    )


def _make_tcpost(n_pad):
    BLK = 512
    grid = (n_pad // BLK,)

    def body(acc_ref, acc16_ref, vs_ref, y_ref, t_ref, aux_ref,
             wet_ref, be_ref, wt_ref, bias_ref, out_ref):
        u = acc_ref[0] + acc_ref[1]
        v16 = acc16_ref[0] + acc16_ref[1]
        vs = jnp.sum(vs_ref[...], axis=0, keepdims=True)
        vs_c = jnp.transpose(vs, (1, 0))
        y = y_ref[...]
        t = t_ref[...]
        la = aux_ref[:, :16]
        dis2 = aux_ref[:, 16:17]
        a = dis2 * (u + y)
        b = t * (v16 + t * la)
        s = dis2 * (vs_c + dis2)
        pre = (a + jnp.dot(b, wet_ref[...], preferred_element_type=F32)
               + s * be_ref[...])
        out_ref[...] = (jnp.dot(pre, wt_ref[...], preferred_element_type=F32)
                        + bias_ref[...])

    return pl.pallas_call(
        body,
        grid=grid,
        in_specs=[
            pl.BlockSpec((NC, BLK, 128), lambda i: (0, i, 0)),
            pl.BlockSpec((NC, BLK, 16), lambda i: (0, i, 0)),
            pl.BlockSpec((NW, BLK), lambda i: (0, i)),
            pl.BlockSpec((BLK, 128), lambda i: (i, 0)),
            pl.BlockSpec((BLK, 16), lambda i: (i, 0)),
            pl.BlockSpec((BLK, 32), lambda i: (i, 0)),
            pl.BlockSpec((16, 128), lambda i: (0, 0)),
            pl.BlockSpec((1, 128), lambda i: (0, 0)),
            pl.BlockSpec((128, 128), lambda i: (0, 0)),
            pl.BlockSpec((1, 128), lambda i: (0, 0)),
        ],
        out_specs=pl.BlockSpec((BLK, 128), lambda i: (i, 0)),
        out_shape=jax.ShapeDtypeStruct((n_pad, 128), F32),
    )


def kernel(x, edge_index, edge_attr, W, We, be, bias):
    n, d_in = x.shape
    E = edge_index.shape[1]
    n_pad = -(-(n + 1) // 1024) * 1024

    ei = edge_index.astype(I32)
    rows = ei[0]
    cols = ei[1]
    attr = edge_attr.astype(F32)
    x_pad = jnp.pad(x.astype(F32), ((0, n_pad - n), (0, 0)))

    sa, sl, cnt = _make_pass1(E, n, n_pad)(rows, cols, attr)
    y, t, dis2t, aux = _make_tcmid(n_pad)(sa, sl, cnt, x_pad)
    acc128 = _make_pass2a(E, n, n_pad)(ei, y)
    acc16, vs = _make_pass2b(E, n, n_pad)(rows, cols, attr, t, dis2t[0])
    out = _make_tcpost(n_pad)(acc128, acc16, vs, y, t, aux,
                              We.T.astype(F32), be.astype(F32)[None],
                              W.T.astype(F32), bias.astype(F32)[None])
    return out[:n]


# speedup vs baseline: 1.3692x; 1.0124x over previous
import jax
import jax.numpy as jnp
from jax import lax
from jax.experimental import pallas as pl
from jax.experimental.pallas import tpu as pltpu
from jax.experimental.pallas import tpu_sc as plsc

F32 = jnp.float32
I32 = jnp.int32

NC = 2
NS = 16
L = 16
NW = NC * NS
SUB = 128

_SC_PARAMS = dict(
    compiler_params=pltpu.CompilerParams(
        needs_layout_passes=False, use_tc_tiling_on_sc=False),
)


def _sc_mesh():
    return plsc.VectorSubcoreMesh(core_axis_name="c", subcore_axis_name="s")


def _make_pass1(E, n, n_pad):
    CHUNK = 512
    NSUB = CHUNK // SUB
    GP = CHUNK // L
    n_chunks = E // CHUNK
    max_chunks = -(-n_chunks // NW)
    zrows = n_pad // NS

    def body(rows_h, cols_h, attr_h,
             sa_h, sl_h, cnt_h,
             r_v, c_v, a_v, ia_v, ib_v,
             cnt_v, slc_v, cra_v, z_v,
             sa_s, sl_s):
        cid = lax.axis_index("c")
        sid = lax.axis_index("s")
        wid = cid * NS + sid
        zero16 = jnp.zeros((L,), F32)
        ones16 = jnp.ones((L,), F32)
        trash16 = jnp.full((L,), n, I32)

        def _zz(i, carry):
            z_v[i, :] = zero16
            return carry
        lax.fori_loop(0, 64, _zz, 0)

        def _zc(i, carry):
            cnt_v[pl.ds(i * L, L)] = zero16
            slc_v[pl.ds(i * L, L)] = zero16
            cra_v[pl.ds(i * L, L)] = zero16
            return carry
        lax.fori_loop(0, n_pad // L, _zc, 0)

        def _zs(i, carry):
            r0 = sid * zrows + i * 64
            pltpu.sync_copy(z_v, sa_s.at[pl.ds(r0, 64)])
            pltpu.sync_copy(z_v, sl_s.at[pl.ds(r0, 64)])
            return carry
        lax.fori_loop(0, zrows // 64, _zs, 0)
        plsc.subcore_barrier()

        def chunk_body(k, carry):
            i = wid + k * NW

            @pl.when(i < n_chunks)
            def _():
                base = i * CHUNK
                pltpu.sync_copy(rows_h.at[pl.ds(base, CHUNK)], r_v)
                pltpu.sync_copy(cols_h.at[pl.ds(base, CHUNK)], c_v)
                pltpu.sync_copy(attr_h.at[pl.ds(base, CHUNK)], a_v)

                def grp(g, carry2):
                    r16 = r_v[pl.ds(g * L, L)]
                    c16 = c_v[pl.ds(g * L, L)]
                    m = r16 != c16
                    mf = jnp.where(m, 1.0, 0.0).astype(F32)
                    nmf = (1.0 - mf).astype(F32)
                    cb = jnp.where(m, trash16, c16)
                    ia_v[0, pl.ds(g * L, L)] = c16
                    ib_v[0, pl.ds(g * L, L)] = cb
                    plsc.addupdate_scatter(cnt_v, [c16], ones16)
                    plsc.addupdate_scatter(slc_v, [c16], nmf)
                    plsc.addupdate_scatter(cra_v, [r16], mf)
                    return carry2
                lax.fori_loop(0, GP, grp, 0)
                pltpu.sync_copy(a_v, sa_s.at[ia_v.at[0]], add=True)
                pltpu.sync_copy(a_v, sl_s.at[ib_v.at[0]], add=True)
            return carry
        lax.fori_loop(0, max_chunks, chunk_body, 0)
        plsc.subcore_barrier()

        r0 = sid * zrows
        pltpu.sync_copy(sa_s.at[pl.ds(r0, zrows)], sa_h.at[cid, pl.ds(r0, zrows)])
        pltpu.sync_copy(sl_s.at[pl.ds(r0, zrows)], sl_h.at[cid, pl.ds(r0, zrows)])
        pltpu.sync_copy(cnt_v, cnt_h.at[0, wid])
        pltpu.sync_copy(slc_v, cnt_h.at[1, wid])
        pltpu.sync_copy(cra_v, cnt_h.at[2, wid])

    return pl.kernel(
        body,
        out_type=(jax.ShapeDtypeStruct((NC, n_pad, 16), F32),
                  jax.ShapeDtypeStruct((NC, n_pad, 16), F32),
                  jax.ShapeDtypeStruct((3, NW, n_pad), F32)),
        mesh=_sc_mesh(),
        scratch_types=[
            pltpu.VMEM((CHUNK,), I32),
            pltpu.VMEM((CHUNK,), I32),
            pltpu.VMEM((CHUNK, 16), F32),
            pltpu.VMEM((1, CHUNK), I32),
            pltpu.VMEM((1, CHUNK), I32),
            pltpu.VMEM((n_pad,), F32),
            pltpu.VMEM((n_pad,), F32),
            pltpu.VMEM((n_pad,), F32),
            pltpu.VMEM((64, 16), F32),
            pltpu.VMEM_SHARED((n_pad, 16), F32),
            pltpu.VMEM_SHARED((n_pad, 16), F32),
        ],
        **_SC_PARAMS,
    )


def _make_pass2a(E, n, n_pad):
    CHUNK = SUB
    GP = CHUNK // L
    n_chunks = E // CHUNK
    max_chunks = -(-n_chunks // NW)
    if max_chunks % 2:
        max_chunks += 1
    zrows = n_pad // NS

    def body(ei_h, y_h,
             acc_h,
             i0_v, i1_v, ir0_v, ir1_v, ic0_v, ic1_v, y0_v, y1_v, z_v,
             acc_s, sem0, sem1):
        cid = lax.axis_index("c")
        sid = lax.axis_index("s")
        wid = cid * NS + sid
        zero16 = jnp.zeros((L,), F32)
        trash16 = jnp.full((L,), n, I32)
        bufs = ((i0_v, ir0_v, ic0_v, y0_v, sem0),
                (i1_v, ir1_v, ic1_v, y1_v, sem1))

        def _zz(i, carry):
            for q in range(128 // L):
                z_v[i, pl.ds(q * L, L)] = zero16
            return carry
        lax.fori_loop(0, 32, _zz, 0)

        def _zs(i, carry):
            pltpu.sync_copy(z_v, acc_s.at[pl.ds(sid * zrows + i * 32, 32)])
            return carry
        lax.fori_loop(0, zrows // 32, _zs, 0)
        plsc.subcore_barrier()

        def fire(k, buf):
            i_v, ir_v, ic_v, y_v, sem = buf
            i = wid + k * NW

            @pl.when(i < n_chunks)
            def _():
                base = i * CHUNK
                pltpu.sync_copy(ei_h.at[:, pl.ds(base, CHUNK)], i_v)

                def grp(g, carry2):
                    r16 = i_v[0, pl.ds(g * L, L)]
                    c16 = i_v[1, pl.ds(g * L, L)]
                    csc = jnp.where(r16 != c16, c16, trash16)
                    ir_v[0, pl.ds(g * L, L)] = r16
                    ic_v[0, pl.ds(g * L, L)] = csc
                    return carry2
                lax.fori_loop(0, GP, grp, 0)
                pltpu.async_copy(y_h.at[ir_v.at[0]], y_v, sem)

        def drain_scatter(k, buf):
            i_v, ir_v, ic_v, y_v, sem = buf
            i = wid + k * NW

            @pl.when(i < n_chunks)
            def _():
                pltpu.make_async_copy(y_h.at[ir_v.at[0]], y_v, sem).wait()
                pltpu.sync_copy(y_v, acc_s.at[ic_v.at[0]], add=True)

        fire(0, bufs[0])

        def chunk_body(k2, carry):
            k = 2 * k2
            fire(k + 1, bufs[1])
            drain_scatter(k, bufs[0])
            fire(k + 2, bufs[0])
            drain_scatter(k + 1, bufs[1])
            return carry
        lax.fori_loop(0, max_chunks // 2, chunk_body, 0)
        plsc.subcore_barrier()

        r0 = sid * zrows
        pltpu.sync_copy(acc_s.at[pl.ds(r0, zrows)],
                        acc_h.at[cid, pl.ds(r0, zrows)])

    return pl.kernel(
        body,
        out_type=jax.ShapeDtypeStruct((NC, n_pad, 128), F32),
        mesh=_sc_mesh(),
        scratch_types=[
            pltpu.VMEM((2, CHUNK), I32),
            pltpu.VMEM((2, CHUNK), I32),
            pltpu.VMEM((1, SUB), I32),
            pltpu.VMEM((1, SUB), I32),
            pltpu.VMEM((1, SUB), I32),
            pltpu.VMEM((1, SUB), I32),
            pltpu.VMEM((CHUNK, 128), F32),
            pltpu.VMEM((CHUNK, 128), F32),
            pltpu.VMEM((32, 128), F32),
            pltpu.VMEM_SHARED((n_pad, 128), F32),
            pltpu.SemaphoreType.DMA,
            pltpu.SemaphoreType.DMA,
        ],
        **_SC_PARAMS,
    )


def _make_pass2b(E, n, n_pad):
    CHUNK = 512
    NSUB = CHUNK // SUB
    GP = CHUNK // L
    n_chunks = E // CHUNK
    max_chunks = -(-n_chunks // NW)
    zrows = n_pad // NS

    def body(rows_h, cols_h, attr_h, t_h, dis2_h,
             acc_h, vs_h,
             r_v, c_v, a_v, ir_v, ic_v, t_v, d2_v, vs_v, z_v,
             acc_s, sem):
        cid = lax.axis_index("c")
        sid = lax.axis_index("s")
        wid = cid * NS + sid
        zero16 = jnp.zeros((L,), F32)
        trash16 = jnp.full((L,), n, I32)

        pltpu.sync_copy(dis2_h, d2_v)

        def _zz(i, carry):
            z_v[i, :] = zero16
            return carry
        lax.fori_loop(0, 64, _zz, 0)

        def _zc(i, carry):
            vs_v[pl.ds(i * L, L)] = zero16
            return carry
        lax.fori_loop(0, n_pad // L, _zc, 0)

        def _zs(i, carry):
            pltpu.sync_copy(z_v, acc_s.at[pl.ds(sid * zrows + i * 64, 64)])
            return carry
        lax.fori_loop(0, zrows // 64, _zs, 0)
        plsc.subcore_barrier()

        def chunk_body(k, carry):
            i = wid + k * NW

            @pl.when(i < n_chunks)
            def _():
                base = i * CHUNK
                pltpu.sync_copy(rows_h.at[pl.ds(base, CHUNK)], r_v)
                pltpu.sync_copy(cols_h.at[pl.ds(base, CHUNK)], c_v)
                pltpu.sync_copy(attr_h.at[pl.ds(base, CHUNK)], a_v)

                def grp(g, carry2):
                    r16 = r_v[pl.ds(g * L, L)]
                    c16 = c_v[pl.ds(g * L, L)]
                    csc = jnp.where(r16 != c16, c16, trash16)
                    ir_v[0, pl.ds(g * L, L)] = r16
                    ic_v[0, pl.ds(g * L, L)] = csc
                    d2r = plsc.load_gather(d2_v, [r16])
                    plsc.addupdate_scatter(vs_v, [csc], d2r)
                    return carry2
                lax.fori_loop(0, GP, grp, 0)

                pltpu.async_copy(t_h.at[ir_v.at[0]], t_v, sem).wait()

                def grp_mul(g, carry2):
                    for q in range(L):
                        e = g * L + q
                        t_v[e, :] = t_v[e, :] * a_v[e, :]
                    return carry2
                lax.fori_loop(0, GP, grp_mul, 0)

                pltpu.sync_copy(t_v, acc_s.at[ic_v.at[0]], add=True)
            return carry
        lax.fori_loop(0, max_chunks, chunk_body, 0)
        plsc.subcore_barrier()

        r0 = sid * zrows
        pltpu.sync_copy(acc_s.at[pl.ds(r0, zrows)],
                        acc_h.at[cid, pl.ds(r0, zrows)])
        pltpu.sync_copy(vs_v, vs_h.at[wid])

    return pl.kernel(
        body,
        out_type=(jax.ShapeDtypeStruct((NC, n_pad, 16), F32),
                  jax.ShapeDtypeStruct((NW, n_pad), F32)),
        mesh=_sc_mesh(),
        scratch_types=[
            pltpu.VMEM((CHUNK,), I32),
            pltpu.VMEM((CHUNK,), I32),
            pltpu.VMEM((CHUNK, 16), F32),
            pltpu.VMEM((1, CHUNK), I32),
            pltpu.VMEM((1, CHUNK), I32),
            pltpu.VMEM((CHUNK, 16), F32),
            pltpu.VMEM((n,), F32),
            pltpu.VMEM((n_pad,), F32),
            pltpu.VMEM((64, 16), F32),
            pltpu.VMEM_SHARED((n_pad, 16), F32),
            pltpu.SemaphoreType.DMA,
        ],
        **_SC_PARAMS,
    )


def _make_tcmid(n, n_pad):
    BLK = 512
    grid = (-(-n // BLK),)

    def body(sa_ref, sl_ref, cnt_ref, x_ref, y_ref, t_ref, dis2_ref, aux_ref):
        sa = sa_ref[0] + sa_ref[1]
        sl = sl_ref[0] + sl_ref[1]
        cnt = jnp.sum(cnt_ref[...], axis=1)
        cnt_t = jnp.transpose(cnt, (1, 0))
        cnt_all = cnt_t[:, 0:1]
        slc = cnt_t[:, 1:2]
        cra = cnt_t[:, 2:3]
        la_mean = sa / jnp.maximum(cnt_all, 1.0)
        la_sl = sl / jnp.maximum(slc, 1.0)
        flag = jnp.minimum(slc, 1.0)
        la = flag * la_sl + (1.0 - flag) * la_mean
        degw = sa - sl + la
        dis = jnp.where(degw > 0.0, lax.rsqrt(jnp.maximum(degw, 1e-30)), 0.0)
        dis2 = lax.rsqrt(cra + 1.0)
        y_ref[...] = dis2 * x_ref[...]
        t_ref[...] = dis2 * dis
        dis2_ref[...] = lax.rsqrt(cnt[2:3, :] + 1.0)
        aux_ref[:, :16] = la
        aux_ref[:, 16:] = jnp.broadcast_to(dis2, (BLK, 16))

    return pl.pallas_call(
        body,
        grid=grid,
        in_specs=[
            pl.BlockSpec((NC, BLK, 16), lambda i: (0, i, 0)),
            pl.BlockSpec((NC, BLK, 16), lambda i: (0, i, 0)),
            pl.BlockSpec((3, NW, BLK), lambda i: (0, 0, i)),
            pl.BlockSpec((BLK, 128), lambda i: (i, 0)),
        ],
        out_specs=[
            pl.BlockSpec((BLK, 128), lambda i: (i, 0)),
            pl.BlockSpec((BLK, 16), lambda i: (i, 0)),
            pl.BlockSpec((1, BLK), lambda i: (0, i)),
            pl.BlockSpec((BLK, 32), lambda i: (i, 0)),
        ],
        out_shape=(jax.ShapeDtypeStruct((n, 128), F32),
                   jax.ShapeDtypeStruct((n, 16), F32),
                   jax.ShapeDtypeStruct((1, n), F32),
                   jax.ShapeDtypeStruct((n, 32), F32)),
    )


def _make_tcpost(n, n_pad):
    BLK = 512
    grid = (-(-n // BLK),)

    def body(acc_ref, acc16_ref, vs_ref, y_ref, t_ref, aux_ref,
             wet_ref, be_ref, wt_ref, bias_ref, out_ref):
        u = acc_ref[0] + acc_ref[1]
        v16 = acc16_ref[0] + acc16_ref[1]
        vs = jnp.sum(vs_ref[...], axis=0, keepdims=True)
        vs_c = jnp.transpose(vs, (1, 0))
        y = y_ref[...]
        t = t_ref[...]
        la = aux_ref[:, :16]
        dis2 = aux_ref[:, 16:17]
        a = dis2 * (u + y)
        b = t * (v16 + t * la)
        s = dis2 * (vs_c + dis2)
        pre = (a + jnp.dot(b, wet_ref[...], preferred_element_type=F32)
               + s * be_ref[...])
        out_ref[...] = (jnp.dot(pre, wt_ref[...], preferred_element_type=F32)
                        + bias_ref[...])

    return pl.pallas_call(
        body,
        grid=grid,
        in_specs=[
            pl.BlockSpec((NC, BLK, 128), lambda i: (0, i, 0)),
            pl.BlockSpec((NC, BLK, 16), lambda i: (0, i, 0)),
            pl.BlockSpec((NW, BLK), lambda i: (0, i)),
            pl.BlockSpec((BLK, 128), lambda i: (i, 0)),
            pl.BlockSpec((BLK, 16), lambda i: (i, 0)),
            pl.BlockSpec((BLK, 32), lambda i: (i, 0)),
            pl.BlockSpec((16, 128), lambda i: (0, 0)),
            pl.BlockSpec((1, 128), lambda i: (0, 0)),
            pl.BlockSpec((128, 128), lambda i: (0, 0)),
            pl.BlockSpec((1, 128), lambda i: (0, 0)),
        ],
        out_specs=pl.BlockSpec((BLK, 128), lambda i: (i, 0)),
        out_shape=jax.ShapeDtypeStruct((n, 128), F32),
    )


def kernel(x, edge_index, edge_attr, W, We, be, bias):
    n, d_in = x.shape
    E = edge_index.shape[1]
    n_pad = -(-(n + 1) // 1024) * 1024

    ei = edge_index.astype(I32)
    rows = ei[0]
    cols = ei[1]
    attr = edge_attr.astype(F32)
    x32 = x.astype(F32)

    sa, sl, cnt = _make_pass1(E, n, n_pad)(rows, cols, attr)
    y, t, dis2t, aux = _make_tcmid(n, n_pad)(sa, sl, cnt, x32)
    acc128 = _make_pass2a(E, n, n_pad)(ei, y)
    acc16, vs = _make_pass2b(E, n, n_pad)(rows, cols, attr, t, dis2t[0])
    return _make_tcpost(n, n_pad)(acc128, acc16, vs, y, t, aux,
                                  We.T.astype(F32), be.astype(F32)[None],
                                  W.T.astype(F32), bias.astype(F32)[None])


# speedup vs baseline: 1.5571x; 1.1372x over previous
import jax
import jax.numpy as jnp
from jax import lax
from jax.experimental import pallas as pl
from jax.experimental.pallas import tpu as pltpu
from jax.experimental.pallas import tpu_sc as plsc

F32 = jnp.float32
I32 = jnp.int32

NC = 2
NS = 16
L = 16
NW = NC * NS

_SC_PARAMS = dict(
    compiler_params=pltpu.CompilerParams(
        needs_layout_passes=False, use_tc_tiling_on_sc=False),
)


def _sc_mesh():
    return plsc.VectorSubcoreMesh(core_axis_name="c", subcore_axis_name="s")


def _make_pass1(E, n, n_pad):
    CHUNK = 512
    GP = CHUNK // L
    n_chunks = E // CHUNK
    max_chunks = -(-n_chunks // NW)
    zrows = n_pad // NS

    def body(ei_h, attr_h,
             sa_h, sl_h, cnt_h,
             e0, e1, e2, a0, a1, a2,
             ia0, ia1, ia2, ib0, ib1, ib2,
             cnt_v, slc_v, cra_v, z_v,
             sa_s, sl_s,
             si0, si1, si2, sm0, sm1, sm2, ss0, ss1, ss2):
        cid = lax.axis_index("c")
        sid = lax.axis_index("s")
        wid = cid * NS + sid
        zero16 = jnp.zeros((L,), F32)
        ones16 = jnp.ones((L,), F32)
        trash16 = jnp.full((L,), n, I32)
        ebufs = (e0, e1, e2)
        abufs = (a0, a1, a2)
        iabufs = (ia0, ia1, ia2)
        ibbufs = (ib0, ib1, ib2)
        sis = (si0, si1, si2)
        sms = (sm0, sm1, sm2)
        sss = (ss0, ss1, ss2)

        def _zz(i, carry):
            z_v[i, :] = zero16
            return carry
        lax.fori_loop(0, 64, _zz, 0)

        def _zc(i, carry):
            cnt_v[pl.ds(i * L, L)] = zero16
            slc_v[pl.ds(i * L, L)] = zero16
            cra_v[pl.ds(i * L, L)] = zero16
            return carry
        lax.fori_loop(0, n_pad // L, _zc, 0)

        def _zs(i, carry):
            r0 = sid * zrows + i * 64
            pltpu.sync_copy(z_v, sa_s.at[pl.ds(r0, 64)])
            pltpu.sync_copy(z_v, sl_s.at[pl.ds(r0, 64)])
            return carry
        lax.fori_loop(0, zrows // 64, _zs, 0)
        plsc.subcore_barrier()

        def fire_inputs(k, q):
            i = wid + k * NW

            @pl.when(i < n_chunks)
            def _():
                base = i * CHUNK
                pltpu.async_copy(ei_h.at[:, pl.ds(base, CHUNK)], ebufs[q], sis[q])
                pltpu.async_copy(attr_h.at[pl.ds(base, CHUNK)], abufs[q], sms[q])

        def drain_scatters(k, q):
            i = wid + k * NW

            @pl.when((k >= 0) & (i < n_chunks))
            def _():
                pltpu.make_async_copy(abufs[q], sa_s.at[iabufs[q].at[0]], sss[q]).wait()
                pltpu.make_async_copy(abufs[q], sl_s.at[ibbufs[q].at[0]], sss[q]).wait()

        def process(k, q):
            i = wid + k * NW

            @pl.when(i < n_chunks)
            def _():
                base = i * CHUNK
                pltpu.make_async_copy(ei_h.at[:, pl.ds(base, CHUNK)], ebufs[q], sis[q]).wait()
                pltpu.make_async_copy(attr_h.at[pl.ds(base, CHUNK)], abufs[q], sms[q]).wait()

                def grp(g, carry2):
                    r16 = ebufs[q][0, pl.ds(g * L, L)]
                    c16 = ebufs[q][1, pl.ds(g * L, L)]
                    m = r16 != c16
                    mf = jnp.where(m, 1.0, 0.0).astype(F32)
                    nmf = (1.0 - mf).astype(F32)
                    iabufs[q][0, pl.ds(g * L, L)] = c16
                    ibbufs[q][0, pl.ds(g * L, L)] = jnp.where(m, trash16, c16)
                    plsc.addupdate_scatter(cnt_v, [c16], ones16)
                    plsc.addupdate_scatter(slc_v, [c16], nmf)
                    plsc.addupdate_scatter(cra_v, [r16], mf)
                    return carry2
                lax.fori_loop(0, GP, grp, 0)
                pltpu.async_copy(abufs[q], sa_s.at[iabufs[q].at[0]], sss[q], add=True)
                pltpu.async_copy(abufs[q], sl_s.at[ibbufs[q].at[0]], sss[q], add=True)

        def step(k, p):
            q = (p + 1) % 3
            drain_scatters(k - 2, q)
            fire_inputs(k + 1, q)
            process(k, p)

        fire_inputs(0, 0)

        def loop3(k3, carry):
            k = 3 * k3
            step(k, 0)
            step(k + 1, 1)
            step(k + 2, 2)
            return carry
        lax.fori_loop(0, (max_chunks + 5) // 3, loop3, 0)
        plsc.subcore_barrier()

        r0 = sid * zrows
        pltpu.sync_copy(sa_s.at[pl.ds(r0, zrows)], sa_h.at[cid, pl.ds(r0, zrows)])
        pltpu.sync_copy(sl_s.at[pl.ds(r0, zrows)], sl_h.at[cid, pl.ds(r0, zrows)])
        pltpu.sync_copy(cnt_v, cnt_h.at[0, wid])
        pltpu.sync_copy(slc_v, cnt_h.at[1, wid])
        pltpu.sync_copy(cra_v, cnt_h.at[2, wid])

    return pl.kernel(
        body,
        out_type=(jax.ShapeDtypeStruct((NC, n_pad, 16), F32),
                  jax.ShapeDtypeStruct((NC, n_pad, 16), F32),
                  jax.ShapeDtypeStruct((3, NW, n_pad), F32)),
        mesh=_sc_mesh(),
        scratch_types=(
            [pltpu.VMEM((2, CHUNK), I32)] * 3
            + [pltpu.VMEM((CHUNK, 16), F32)] * 3
            + [pltpu.VMEM((1, CHUNK), I32)] * 6
            + [pltpu.VMEM((n_pad,), F32)] * 3
            + [pltpu.VMEM((64, 16), F32),
               pltpu.VMEM_SHARED((n_pad, 16), F32),
               pltpu.VMEM_SHARED((n_pad, 16), F32)]
            + [pltpu.SemaphoreType.DMA] * 9
        ),
        **_SC_PARAMS,
    )


def _make_pass2a(E, n, n_pad):
    CHUNK = 128
    GP = CHUNK // L
    n_chunks = E // CHUNK
    max_chunks = -(-n_chunks // NW)
    zrows = n_pad // NS

    def body(ei_h, y_h,
             acc_h,
             e0, e1, ir0, ir1, ic0, ic1, y0, y1, z_v,
             acc_s, si0, si1, sg0, sg1, ss0, ss1):
        cid = lax.axis_index("c")
        sid = lax.axis_index("s")
        wid = cid * NS + sid
        zero16 = jnp.zeros((L,), F32)
        trash16 = jnp.full((L,), n, I32)
        ebufs = (e0, e1)
        irbufs = (ir0, ir1)
        icbufs = (ic0, ic1)
        ybufs = (y0, y1)
        sis = (si0, si1)
        sgs = (sg0, sg1)
        sss = (ss0, ss1)

        def _zz(i, carry):
            for q in range(128 // L):
                z_v[i, pl.ds(q * L, L)] = zero16
            return carry
        lax.fori_loop(0, 32, _zz, 0)

        def _zs(i, carry):
            pltpu.sync_copy(z_v, acc_s.at[pl.ds(sid * zrows + i * 32, 32)])
            return carry
        lax.fori_loop(0, zrows // 32, _zs, 0)
        plsc.subcore_barrier()

        def fire_idx(k, p):
            i = wid + k * NW

            @pl.when(i < n_chunks)
            def _():
                pltpu.async_copy(ei_h.at[:, pl.ds(i * CHUNK, CHUNK)],
                                 ebufs[p], sis[p])

        def drain_scatter(k, p):
            i = wid + k * NW

            @pl.when((k >= 0) & (i < n_chunks))
            def _():
                pltpu.make_async_copy(ybufs[p], acc_s.at[icbufs[p].at[0]],
                                      sss[p]).wait()

        def gather(k, p):
            i = wid + k * NW

            @pl.when(i < n_chunks)
            def _():
                pltpu.make_async_copy(ei_h.at[:, pl.ds(i * CHUNK, CHUNK)],
                                      ebufs[p], sis[p]).wait()

                def grp(g, carry2):
                    r16 = ebufs[p][0, pl.ds(g * L, L)]
                    c16 = ebufs[p][1, pl.ds(g * L, L)]
                    irbufs[p][0, pl.ds(g * L, L)] = r16
                    icbufs[p][0, pl.ds(g * L, L)] = jnp.where(
                        r16 != c16, c16, trash16)
                    return carry2
                lax.fori_loop(0, GP, grp, 0)
                pltpu.async_copy(y_h.at[irbufs[p].at[0]], ybufs[p], sgs[p])

        def scatter(k, p):
            i = wid + k * NW

            @pl.when((k >= 0) & (i < n_chunks))
            def _():
                pltpu.make_async_copy(y_h.at[irbufs[p].at[0]], ybufs[p],
                                      sgs[p]).wait()
                pltpu.async_copy(ybufs[p], acc_s.at[icbufs[p].at[0]],
                                 sss[p], add=True)

        fire_idx(0, 0)

        def loop2(k2, carry):
            k = 2 * k2
            drain_scatter(k - 2, 0)
            gather(k, 0)
            fire_idx(k + 1, 1)
            scatter(k - 1, 1)
            drain_scatter(k - 1, 1)
            gather(k + 1, 1)
            fire_idx(k + 2, 0)
            scatter(k, 0)
            return carry
        lax.fori_loop(0, (max_chunks + 4) // 2, loop2, 0)
        plsc.subcore_barrier()

        r0 = sid * zrows
        pltpu.sync_copy(acc_s.at[pl.ds(r0, zrows)],
                        acc_h.at[cid, pl.ds(r0, zrows)])

    return pl.kernel(
        body,
        out_type=jax.ShapeDtypeStruct((NC, n_pad, 128), F32),
        mesh=_sc_mesh(),
        scratch_types=(
            [pltpu.VMEM((2, CHUNK), I32)] * 2
            + [pltpu.VMEM((1, CHUNK), I32)] * 4
            + [pltpu.VMEM((CHUNK, 128), F32)] * 2
            + [pltpu.VMEM((32, 128), F32),
               pltpu.VMEM_SHARED((n_pad, 128), F32)]
            + [pltpu.SemaphoreType.DMA] * 6
        ),
        **_SC_PARAMS,
    )


def _make_pass2b(E, n, n_pad):
    CHUNK = 512
    GP = CHUNK // L
    n_chunks = E // CHUNK
    max_chunks = -(-n_chunks // NW)
    zrows = n_pad // NS

    def body(ei_h, attr_h, t_h, dis2_h,
             acc_h, vs_h,
             e0, e1, a0, a1, t0, t1, ir0, ir1, ic0, ic1,
             d2_v, vs_v, z_v,
             acc_s,
             si0, si1, sm0, sm1, sg0, sg1, ss0, ss1):
        cid = lax.axis_index("c")
        sid = lax.axis_index("s")
        wid = cid * NS + sid
        zero16 = jnp.zeros((L,), F32)
        trash16 = jnp.full((L,), n, I32)
        ebufs = (e0, e1)
        abufs = (a0, a1)
        tbufs = (t0, t1)
        irbufs = (ir0, ir1)
        icbufs = (ic0, ic1)
        sis = (si0, si1)
        sms = (sm0, sm1)
        sgs = (sg0, sg1)
        sss = (ss0, ss1)

        pltpu.sync_copy(dis2_h.at[0], d2_v)

        def _zz(i, carry):
            z_v[i, :] = zero16
            return carry
        lax.fori_loop(0, 64, _zz, 0)

        def _zc(i, carry):
            vs_v[pl.ds(i * L, L)] = zero16
            return carry
        lax.fori_loop(0, n_pad // L, _zc, 0)

        def _zs(i, carry):
            pltpu.sync_copy(z_v, acc_s.at[pl.ds(sid * zrows + i * 64, 64)])
            return carry
        lax.fori_loop(0, zrows // 64, _zs, 0)
        plsc.subcore_barrier()

        def fire_inputs(k, p):
            i = wid + k * NW

            @pl.when(i < n_chunks)
            def _():
                base = i * CHUNK
                pltpu.async_copy(ei_h.at[:, pl.ds(base, CHUNK)], ebufs[p], sis[p])
                pltpu.async_copy(attr_h.at[pl.ds(base, CHUNK)], abufs[p], sms[p])

        def drain_scatter(k, p):
            i = wid + k * NW

            @pl.when((k >= 0) & (i < n_chunks))
            def _():
                pltpu.make_async_copy(tbufs[p], acc_s.at[icbufs[p].at[0]],
                                      sss[p]).wait()

        def gather(k, p):
            i = wid + k * NW

            @pl.when(i < n_chunks)
            def _():
                base = i * CHUNK
                pltpu.make_async_copy(ei_h.at[:, pl.ds(base, CHUNK)],
                                      ebufs[p], sis[p]).wait()

                def grp(g, carry2):
                    r16 = ebufs[p][0, pl.ds(g * L, L)]
                    c16 = ebufs[p][1, pl.ds(g * L, L)]
                    csc = jnp.where(r16 != c16, c16, trash16)
                    irbufs[p][0, pl.ds(g * L, L)] = r16
                    icbufs[p][0, pl.ds(g * L, L)] = csc
                    d2r = plsc.load_gather(d2_v, [r16])
                    plsc.addupdate_scatter(vs_v, [csc], d2r)
                    return carry2
                lax.fori_loop(0, GP, grp, 0)
                pltpu.async_copy(t_h.at[irbufs[p].at[0]], tbufs[p], sgs[p])

        def mul_scatter(k, p):
            i = wid + k * NW

            @pl.when((k >= 0) & (i < n_chunks))
            def _():
                base = i * CHUNK
                pltpu.make_async_copy(t_h.at[irbufs[p].at[0]], tbufs[p],
                                      sgs[p]).wait()
                pltpu.make_async_copy(attr_h.at[pl.ds(base, CHUNK)],
                                      abufs[p], sms[p]).wait()

                def grp_mul(g, carry2):
                    for q in range(L):
                        e = g * L + q
                        tbufs[p][e, :] = tbufs[p][e, :] * abufs[p][e, :]
                    return carry2
                lax.fori_loop(0, GP, grp_mul, 0)
                pltpu.async_copy(tbufs[p], acc_s.at[icbufs[p].at[0]],
                                 sss[p], add=True)

        fire_inputs(0, 0)

        def loop2(k2, carry):
            k = 2 * k2
            drain_scatter(k - 2, 0)
            gather(k, 0)
            mul_scatter(k - 1, 1)
            fire_inputs(k + 1, 1)
            drain_scatter(k - 1, 1)
            gather(k + 1, 1)
            mul_scatter(k, 0)
            fire_inputs(k + 2, 0)
            return carry
        lax.fori_loop(0, (max_chunks + 4) // 2, loop2, 0)
        plsc.subcore_barrier()

        r0 = sid * zrows
        pltpu.sync_copy(acc_s.at[pl.ds(r0, zrows)],
                        acc_h.at[cid, pl.ds(r0, zrows)])
        pltpu.sync_copy(vs_v, vs_h.at[wid])

    return pl.kernel(
        body,
        out_type=(jax.ShapeDtypeStruct((NC, n_pad, 16), F32),
                  jax.ShapeDtypeStruct((NW, n_pad), F32)),
        mesh=_sc_mesh(),
        scratch_types=(
            [pltpu.VMEM((2, CHUNK), I32)] * 2
            + [pltpu.VMEM((CHUNK, 16), F32)] * 4
            + [pltpu.VMEM((1, CHUNK), I32)] * 4
            + [pltpu.VMEM((n,), F32),
               pltpu.VMEM((n_pad,), F32),
               pltpu.VMEM((64, 16), F32),
               pltpu.VMEM_SHARED((n_pad, 16), F32)]
            + [pltpu.SemaphoreType.DMA] * 8
        ),
        **_SC_PARAMS,
    )


def _make_tcmid(n, n_pad):
    BLK = 512
    grid = (-(-n // BLK),)

    def body(sa_ref, sl_ref, cnt_ref, x_ref, y_ref, t_ref, dis2_ref, aux_ref):
        sa = sa_ref[0] + sa_ref[1]
        sl = sl_ref[0] + sl_ref[1]
        cnt = jnp.sum(cnt_ref[...], axis=1)
        cnt_t = jnp.transpose(cnt, (1, 0))
        cnt_all = cnt_t[:, 0:1]
        slc = cnt_t[:, 1:2]
        cra = cnt_t[:, 2:3]
        la_mean = sa / jnp.maximum(cnt_all, 1.0)
        la_sl = sl / jnp.maximum(slc, 1.0)
        flag = jnp.minimum(slc, 1.0)
        la = flag * la_sl + (1.0 - flag) * la_mean
        degw = sa - sl + la
        dis = jnp.where(degw > 0.0, lax.rsqrt(jnp.maximum(degw, 1e-30)), 0.0)
        dis2 = lax.rsqrt(cra + 1.0)
        y_ref[...] = dis2 * x_ref[...]
        t_ref[...] = dis2 * dis
        dis2_ref[...] = lax.rsqrt(cnt[2:3, :] + 1.0)
        aux_ref[:, :16] = la
        aux_ref[:, 16:] = jnp.broadcast_to(dis2, (BLK, 16))

    return pl.pallas_call(
        body,
        grid=grid,
        in_specs=[
            pl.BlockSpec((NC, BLK, 16), lambda i: (0, i, 0)),
            pl.BlockSpec((NC, BLK, 16), lambda i: (0, i, 0)),
            pl.BlockSpec((3, NW, BLK), lambda i: (0, 0, i)),
            pl.BlockSpec((BLK, 128), lambda i: (i, 0)),
        ],
        out_specs=[
            pl.BlockSpec((BLK, 128), lambda i: (i, 0)),
            pl.BlockSpec((BLK, 16), lambda i: (i, 0)),
            pl.BlockSpec((1, BLK), lambda i: (0, i)),
            pl.BlockSpec((BLK, 32), lambda i: (i, 0)),
        ],
        out_shape=(jax.ShapeDtypeStruct((n, 128), F32),
                   jax.ShapeDtypeStruct((n, 16), F32),
                   jax.ShapeDtypeStruct((1, n), F32),
                   jax.ShapeDtypeStruct((n, 32), F32)),
    )


def _make_tcpost(n, n_pad):
    BLK = 512
    grid = (-(-n // BLK),)

    def body(acc_ref, acc16_ref, vs_ref, y_ref, t_ref, aux_ref,
             wet_ref, be_ref, wt_ref, bias_ref, out_ref):
        u = acc_ref[0] + acc_ref[1]
        v16 = acc16_ref[0] + acc16_ref[1]
        vs = jnp.sum(vs_ref[...], axis=0, keepdims=True)
        vs_c = jnp.transpose(vs, (1, 0))
        y = y_ref[...]
        t = t_ref[...]
        la = aux_ref[:, :16]
        dis2 = aux_ref[:, 16:17]
        a = dis2 * (u + y)
        b = t * (v16 + t * la)
        s = dis2 * (vs_c + dis2)
        pre = (a + jnp.dot(b, wet_ref[...], preferred_element_type=F32)
               + s * be_ref[...])
        out_ref[...] = (jnp.dot(pre, wt_ref[...], preferred_element_type=F32)
                        + bias_ref[...])

    return pl.pallas_call(
        body,
        grid=grid,
        in_specs=[
            pl.BlockSpec((NC, BLK, 128), lambda i: (0, i, 0)),
            pl.BlockSpec((NC, BLK, 16), lambda i: (0, i, 0)),
            pl.BlockSpec((NW, BLK), lambda i: (0, i)),
            pl.BlockSpec((BLK, 128), lambda i: (i, 0)),
            pl.BlockSpec((BLK, 16), lambda i: (i, 0)),
            pl.BlockSpec((BLK, 32), lambda i: (i, 0)),
            pl.BlockSpec((16, 128), lambda i: (0, 0)),
            pl.BlockSpec((1, 128), lambda i: (0, 0)),
            pl.BlockSpec((128, 128), lambda i: (0, 0)),
            pl.BlockSpec((1, 128), lambda i: (0, 0)),
        ],
        out_specs=pl.BlockSpec((BLK, 128), lambda i: (i, 0)),
        out_shape=jax.ShapeDtypeStruct((n, 128), F32),
    )


def kernel(x, edge_index, edge_attr, W, We, be, bias):
    n, d_in = x.shape
    E = edge_index.shape[1]
    n_pad = -(-(n + 1) // 1024) * 1024

    ei = edge_index.astype(I32)
    attr = edge_attr.astype(F32)
    x32 = x.astype(F32)

    sa, sl, cnt = _make_pass1(E, n, n_pad)(ei, attr)
    y, t, dis2t, aux = _make_tcmid(n, n_pad)(sa, sl, cnt, x32)
    acc128 = _make_pass2a(E, n, n_pad)(ei, y)
    acc16, vs = _make_pass2b(E, n, n_pad)(ei, attr, t, dis2t)
    return _make_tcpost(n, n_pad)(acc128, acc16, vs, y, t, aux,
                                  We.T.astype(F32), be.astype(F32)[None],
                                  W.T.astype(F32), bias.astype(F32)[None])


# speedup vs baseline: 2.2423x; 1.4400x over previous
import jax
import jax.numpy as jnp
from jax import lax
from jax.experimental import pallas as pl
from jax.experimental.pallas import tpu as pltpu
from jax.experimental.pallas import tpu_sc as plsc

F32 = jnp.float32
I32 = jnp.int32

NC = 2
NS = 16
L = 16
NW = NC * NS

_SC_PARAMS = dict(
    compiler_params=pltpu.CompilerParams(
        needs_layout_passes=False, use_tc_tiling_on_sc=False),
)


def _sc_mesh():
    return plsc.VectorSubcoreMesh(core_axis_name="c", subcore_axis_name="s")


def _make_pass1(E, n, n_pad):
    CHUNK = 512
    GP = CHUNK // L
    n_chunks = E // CHUNK
    max_chunks = -(-n_chunks // NW)
    zrows = n_pad // NS

    def body(ei_h, attr_h,
             sa_h, sl_h, cnt_h,
             e0, e1, e2, a0, a1, a2,
             ia0, ia1, ia2, ib0, ib1, ib2,
             cnt_v, slc_v, cra_v, z_v,
             sa_s, sl_s,
             si0, si1, si2, sm0, sm1, sm2, ss0, ss1, ss2):
        cid = lax.axis_index("c")
        sid = lax.axis_index("s")
        wid = cid * NS + sid
        zero16 = jnp.zeros((L,), F32)
        ones16 = jnp.ones((L,), F32)
        trash16 = jnp.full((L,), n, I32)
        ebufs = (e0, e1, e2)
        abufs = (a0, a1, a2)
        iabufs = (ia0, ia1, ia2)
        ibbufs = (ib0, ib1, ib2)
        sis = (si0, si1, si2)
        sms = (sm0, sm1, sm2)
        sss = (ss0, ss1, ss2)

        def _zz(i, carry):
            z_v[i, :] = zero16
            return carry
        lax.fori_loop(0, 64, _zz, 0)

        def _zc(i, carry):
            cnt_v[pl.ds(i * L, L)] = zero16
            slc_v[pl.ds(i * L, L)] = zero16
            cra_v[pl.ds(i * L, L)] = zero16
            return carry
        lax.fori_loop(0, n_pad // L, _zc, 0)

        def _zs(i, carry):
            r0 = sid * zrows + i * 64
            pltpu.sync_copy(z_v, sa_s.at[pl.ds(r0, 64)])
            pltpu.sync_copy(z_v, sl_s.at[pl.ds(r0, 64)])
            return carry
        lax.fori_loop(0, zrows // 64, _zs, 0)
        plsc.subcore_barrier()

        def fire_inputs(k, q):
            i = wid + k * NW

            @pl.when(i < n_chunks)
            def _():
                base = i * CHUNK
                pltpu.async_copy(ei_h.at[:, pl.ds(base, CHUNK)], ebufs[q], sis[q])
                pltpu.async_copy(attr_h.at[pl.ds(base, CHUNK)], abufs[q], sms[q])

        def drain_scatters(k, q):
            i = wid + k * NW

            @pl.when((k >= 0) & (i < n_chunks))
            def _():
                pltpu.make_async_copy(abufs[q], sa_s.at[iabufs[q].at[0]], sss[q]).wait()
                pltpu.make_async_copy(
                    abufs[q],
                    sl_s.at[plsc.Indices(ibbufs[q].at[0], ignored_value=n)],
                    sss[q]).wait()

        def process(k, q):
            i = wid + k * NW

            @pl.when(i < n_chunks)
            def _():
                base = i * CHUNK
                pltpu.make_async_copy(ei_h.at[:, pl.ds(base, CHUNK)], ebufs[q], sis[q]).wait()
                pltpu.make_async_copy(attr_h.at[pl.ds(base, CHUNK)], abufs[q], sms[q]).wait()

                def grp(g, carry2):
                    r16 = ebufs[q][0, pl.ds(g * L, L)]
                    c16 = ebufs[q][1, pl.ds(g * L, L)]
                    m = r16 != c16
                    mf = jnp.where(m, 1.0, 0.0).astype(F32)
                    nmf = (1.0 - mf).astype(F32)
                    iabufs[q][0, pl.ds(g * L, L)] = c16
                    ibbufs[q][0, pl.ds(g * L, L)] = jnp.where(m, trash16, c16)
                    plsc.addupdate_scatter(cnt_v, [c16], ones16)
                    plsc.addupdate_scatter(slc_v, [c16], nmf)
                    plsc.addupdate_scatter(cra_v, [r16], mf)
                    return carry2
                lax.fori_loop(0, GP, grp, 0)
                pltpu.async_copy(abufs[q], sa_s.at[iabufs[q].at[0]], sss[q], add=True)
                pltpu.async_copy(
                    abufs[q],
                    sl_s.at[plsc.Indices(ibbufs[q].at[0], ignored_value=n)],
                    sss[q], add=True)

        def step(k, p):
            q = (p + 1) % 3
            drain_scatters(k - 2, q)
            fire_inputs(k + 1, q)
            process(k, p)

        fire_inputs(0, 0)

        def loop3(k3, carry):
            k = 3 * k3
            step(k, 0)
            step(k + 1, 1)
            step(k + 2, 2)
            return carry
        lax.fori_loop(0, (max_chunks + 5) // 3, loop3, 0)
        plsc.subcore_barrier()

        r0 = sid * zrows
        pltpu.sync_copy(sa_s.at[pl.ds(r0, zrows)], sa_h.at[cid, pl.ds(r0, zrows)])
        pltpu.sync_copy(sl_s.at[pl.ds(r0, zrows)], sl_h.at[cid, pl.ds(r0, zrows)])
        pltpu.sync_copy(cnt_v, cnt_h.at[0, wid])
        pltpu.sync_copy(slc_v, cnt_h.at[1, wid])
        pltpu.sync_copy(cra_v, cnt_h.at[2, wid])

    return pl.kernel(
        body,
        out_type=(jax.ShapeDtypeStruct((NC, n_pad, 16), F32),
                  jax.ShapeDtypeStruct((NC, n_pad, 16), F32),
                  jax.ShapeDtypeStruct((3, NW, n_pad), F32)),
        mesh=_sc_mesh(),
        scratch_types=(
            [pltpu.VMEM((2, CHUNK), I32)] * 3
            + [pltpu.VMEM((CHUNK, 16), F32)] * 3
            + [pltpu.VMEM((1, CHUNK), I32)] * 6
            + [pltpu.VMEM((n_pad,), F32)] * 3
            + [pltpu.VMEM((64, 16), F32),
               pltpu.VMEM_SHARED((n_pad, 16), F32),
               pltpu.VMEM_SHARED((n_pad, 16), F32)]
            + [pltpu.SemaphoreType.DMA] * 9
        ),
        **_SC_PARAMS,
    )


def _make_pass2a(E, n, n_pad):
    CHUNK = 128
    GP = CHUNK // L
    n_chunks = E // CHUNK
    max_chunks = -(-n_chunks // NW)
    zrows = n_pad // NS

    def body(ei_h, y_h,
             acc_h,
             e0, e1, ir0, ir1, ic0, ic1, y0, y1, z_v,
             acc_s, si0, si1, sg0, sg1, ss0, ss1):
        cid = lax.axis_index("c")
        sid = lax.axis_index("s")
        wid = cid * NS + sid
        zero16 = jnp.zeros((L,), F32)
        trash16 = jnp.full((L,), n, I32)
        ebufs = (e0, e1)
        irbufs = (ir0, ir1)
        icbufs = (ic0, ic1)
        ybufs = (y0, y1)
        sis = (si0, si1)
        sgs = (sg0, sg1)
        sss = (ss0, ss1)

        def _zz(i, carry):
            for q in range(128 // L):
                z_v[i, pl.ds(q * L, L)] = zero16
            return carry
        lax.fori_loop(0, 32, _zz, 0)

        def _zs(i, carry):
            pltpu.sync_copy(z_v, acc_s.at[pl.ds(sid * zrows + i * 32, 32)])
            return carry
        lax.fori_loop(0, zrows // 32, _zs, 0)
        plsc.subcore_barrier()

        def fire_idx(k, p):
            i = wid + k * NW

            @pl.when(i < n_chunks)
            def _():
                pltpu.async_copy(ei_h.at[:, pl.ds(i * CHUNK, CHUNK)],
                                 ebufs[p], sis[p])

        def drain_scatter(k, p):
            i = wid + k * NW

            @pl.when((k >= 0) & (i < n_chunks))
            def _():
                pltpu.make_async_copy(ybufs[p], acc_s.at[icbufs[p].at[0]],
                                      sss[p]).wait()

        def gather(k, p):
            i = wid + k * NW

            @pl.when(i < n_chunks)
            def _():
                pltpu.make_async_copy(ei_h.at[:, pl.ds(i * CHUNK, CHUNK)],
                                      ebufs[p], sis[p]).wait()

                def grp(g, carry2):
                    r16 = ebufs[p][0, pl.ds(g * L, L)]
                    c16 = ebufs[p][1, pl.ds(g * L, L)]
                    irbufs[p][0, pl.ds(g * L, L)] = r16
                    icbufs[p][0, pl.ds(g * L, L)] = jnp.where(
                        r16 != c16, c16, trash16)
                    return carry2
                lax.fori_loop(0, GP, grp, 0)
                pltpu.async_copy(y_h.at[irbufs[p].at[0]], ybufs[p], sgs[p])

        def scatter(k, p):
            i = wid + k * NW

            @pl.when((k >= 0) & (i < n_chunks))
            def _():
                pltpu.make_async_copy(y_h.at[irbufs[p].at[0]], ybufs[p],
                                      sgs[p]).wait()
                pltpu.async_copy(ybufs[p], acc_s.at[icbufs[p].at[0]],
                                 sss[p], add=True)

        fire_idx(0, 0)

        def loop2(k2, carry):
            k = 2 * k2
            drain_scatter(k - 2, 0)
            gather(k, 0)
            fire_idx(k + 1, 1)
            scatter(k - 1, 1)
            drain_scatter(k - 1, 1)
            gather(k + 1, 1)
            fire_idx(k + 2, 0)
            scatter(k, 0)
            return carry
        lax.fori_loop(0, (max_chunks + 4) // 2, loop2, 0)
        plsc.subcore_barrier()

        r0 = sid * zrows
        pltpu.sync_copy(acc_s.at[pl.ds(r0, zrows)],
                        acc_h.at[cid, pl.ds(r0, zrows)])

    return pl.kernel(
        body,
        out_type=jax.ShapeDtypeStruct((NC, n_pad, 128), F32),
        mesh=_sc_mesh(),
        scratch_types=(
            [pltpu.VMEM((2, CHUNK), I32)] * 2
            + [pltpu.VMEM((1, CHUNK), I32)] * 4
            + [pltpu.VMEM((CHUNK, 128), F32)] * 2
            + [pltpu.VMEM((32, 128), F32),
               pltpu.VMEM_SHARED((n_pad, 128), F32)]
            + [pltpu.SemaphoreType.DMA] * 6
        ),
        **_SC_PARAMS,
    )


def _make_pass2b(E, n, n_pad):
    CHUNK = 512
    GP = CHUNK // L
    n_chunks = E // CHUNK
    max_chunks = -(-n_chunks // NW)
    zrows = n_pad // NS

    def body(ei_h, attr_h, t_h, dis2_h,
             acc_h, vs_h,
             e0, e1, a0, a1, t0, t1, ir0, ir1, ic0, ic1,
             d2_v, vs_v, z_v,
             acc_s,
             si0, si1, sm0, sm1, sg0, sg1, ss0, ss1):
        cid = lax.axis_index("c")
        sid = lax.axis_index("s")
        wid = cid * NS + sid
        zero16 = jnp.zeros((L,), F32)
        trash16 = jnp.full((L,), n, I32)
        ebufs = (e0, e1)
        abufs = (a0, a1)
        tbufs = (t0, t1)
        irbufs = (ir0, ir1)
        icbufs = (ic0, ic1)
        sis = (si0, si1)
        sms = (sm0, sm1)
        sgs = (sg0, sg1)
        sss = (ss0, ss1)

        pltpu.sync_copy(dis2_h.at[0], d2_v)

        def _zz(i, carry):
            z_v[i, :] = zero16
            return carry
        lax.fori_loop(0, 64, _zz, 0)

        def _zc(i, carry):
            vs_v[pl.ds(i * L, L)] = zero16
            return carry
        lax.fori_loop(0, n_pad // L, _zc, 0)

        def _zs(i, carry):
            pltpu.sync_copy(z_v, acc_s.at[pl.ds(sid * zrows + i * 64, 64)])
            return carry
        lax.fori_loop(0, zrows // 64, _zs, 0)
        plsc.subcore_barrier()

        def fire_inputs(k, p):
            i = wid + k * NW

            @pl.when(i < n_chunks)
            def _():
                base = i * CHUNK
                pltpu.async_copy(ei_h.at[:, pl.ds(base, CHUNK)], ebufs[p], sis[p])
                pltpu.async_copy(attr_h.at[pl.ds(base, CHUNK)], abufs[p], sms[p])

        def drain_scatter(k, p):
            i = wid + k * NW

            @pl.when((k >= 0) & (i < n_chunks))
            def _():
                pltpu.make_async_copy(tbufs[p], acc_s.at[icbufs[p].at[0]],
                                      sss[p]).wait()

        def gather(k, p):
            i = wid + k * NW

            @pl.when(i < n_chunks)
            def _():
                base = i * CHUNK
                pltpu.make_async_copy(ei_h.at[:, pl.ds(base, CHUNK)],
                                      ebufs[p], sis[p]).wait()

                def grp(g, carry2):
                    r16 = ebufs[p][0, pl.ds(g * L, L)]
                    c16 = ebufs[p][1, pl.ds(g * L, L)]
                    csc = jnp.where(r16 != c16, c16, trash16)
                    irbufs[p][0, pl.ds(g * L, L)] = r16
                    icbufs[p][0, pl.ds(g * L, L)] = csc
                    d2r = plsc.load_gather(d2_v, [r16])
                    plsc.addupdate_scatter(vs_v, [csc], d2r)
                    return carry2
                lax.fori_loop(0, GP, grp, 0)
                pltpu.async_copy(t_h.at[irbufs[p].at[0]], tbufs[p], sgs[p])

        def mul_scatter(k, p):
            i = wid + k * NW

            @pl.when((k >= 0) & (i < n_chunks))
            def _():
                base = i * CHUNK
                pltpu.make_async_copy(t_h.at[irbufs[p].at[0]], tbufs[p],
                                      sgs[p]).wait()
                pltpu.make_async_copy(attr_h.at[pl.ds(base, CHUNK)],
                                      abufs[p], sms[p]).wait()

                def grp_mul(g, carry2):
                    for q in range(L):
                        e = g * L + q
                        tbufs[p][e, :] = tbufs[p][e, :] * abufs[p][e, :]
                    return carry2
                lax.fori_loop(0, GP, grp_mul, 0)
                pltpu.async_copy(tbufs[p], acc_s.at[icbufs[p].at[0]],
                                 sss[p], add=True)

        fire_inputs(0, 0)

        def loop2(k2, carry):
            k = 2 * k2
            drain_scatter(k - 2, 0)
            gather(k, 0)
            mul_scatter(k - 1, 1)
            fire_inputs(k + 1, 1)
            drain_scatter(k - 1, 1)
            gather(k + 1, 1)
            mul_scatter(k, 0)
            fire_inputs(k + 2, 0)
            return carry
        lax.fori_loop(0, (max_chunks + 4) // 2, loop2, 0)
        plsc.subcore_barrier()

        r0 = sid * zrows
        pltpu.sync_copy(acc_s.at[pl.ds(r0, zrows)],
                        acc_h.at[cid, pl.ds(r0, zrows)])
        pltpu.sync_copy(vs_v, vs_h.at[wid])

    return pl.kernel(
        body,
        out_type=(jax.ShapeDtypeStruct((NC, n_pad, 16), F32),
                  jax.ShapeDtypeStruct((NW, n_pad), F32)),
        mesh=_sc_mesh(),
        scratch_types=(
            [pltpu.VMEM((2, CHUNK), I32)] * 2
            + [pltpu.VMEM((CHUNK, 16), F32)] * 4
            + [pltpu.VMEM((1, CHUNK), I32)] * 4
            + [pltpu.VMEM((n,), F32),
               pltpu.VMEM((n_pad,), F32),
               pltpu.VMEM((64, 16), F32),
               pltpu.VMEM_SHARED((n_pad, 16), F32)]
            + [pltpu.SemaphoreType.DMA] * 8
        ),
        **_SC_PARAMS,
    )


def _make_tcmid(n, n_pad):
    BLK = 512
    grid = (-(-n // BLK),)

    def body(sa_ref, sl_ref, cnt_ref, x_ref, y_ref, t_ref, dis2_ref, aux_ref):
        sa = sa_ref[0] + sa_ref[1]
        sl = sl_ref[0] + sl_ref[1]
        cnt = jnp.sum(cnt_ref[...], axis=1)
        cnt_t = jnp.transpose(cnt, (1, 0))
        cnt_all = cnt_t[:, 0:1]
        slc = cnt_t[:, 1:2]
        cra = cnt_t[:, 2:3]
        la_mean = sa / jnp.maximum(cnt_all, 1.0)
        la_sl = sl / jnp.maximum(slc, 1.0)
        flag = jnp.minimum(slc, 1.0)
        la = flag * la_sl + (1.0 - flag) * la_mean
        degw = sa - sl + la
        dis = jnp.where(degw > 0.0, lax.rsqrt(jnp.maximum(degw, 1e-30)), 0.0)
        dis2 = lax.rsqrt(cra + 1.0)
        y_ref[...] = dis2 * x_ref[...]
        t_ref[...] = dis2 * dis
        dis2_ref[...] = lax.rsqrt(cnt[2:3, :] + 1.0)
        aux_ref[:, :16] = la
        aux_ref[:, 16:] = jnp.broadcast_to(dis2, (BLK, 16))

    return pl.pallas_call(
        body,
        grid=grid,
        in_specs=[
            pl.BlockSpec((NC, BLK, 16), lambda i: (0, i, 0)),
            pl.BlockSpec((NC, BLK, 16), lambda i: (0, i, 0)),
            pl.BlockSpec((3, NW, BLK), lambda i: (0, 0, i)),
            pl.BlockSpec((BLK, 128), lambda i: (i, 0)),
        ],
        out_specs=[
            pl.BlockSpec((BLK, 128), lambda i: (i, 0)),
            pl.BlockSpec((BLK, 16), lambda i: (i, 0)),
            pl.BlockSpec((1, BLK), lambda i: (0, i)),
            pl.BlockSpec((BLK, 32), lambda i: (i, 0)),
        ],
        out_shape=(jax.ShapeDtypeStruct((n, 128), F32),
                   jax.ShapeDtypeStruct((n, 16), F32),
                   jax.ShapeDtypeStruct((1, n), F32),
                   jax.ShapeDtypeStruct((n, 32), F32)),
    )


def _make_tcpost(n, n_pad):
    BLK = 512
    grid = (-(-n // BLK),)

    def body(acc_ref, acc16_ref, vs_ref, y_ref, t_ref, aux_ref,
             we_ref, be_ref, w_ref, bias_ref, out_ref):
        u = acc_ref[0] + acc_ref[1]
        v16 = acc16_ref[0] + acc16_ref[1]
        vs = jnp.sum(vs_ref[...], axis=0, keepdims=True)
        vs_c = jnp.transpose(vs, (1, 0))
        y = y_ref[...]
        t = t_ref[...]
        la = aux_ref[:, :16]
        dis2 = aux_ref[:, 16:17]
        a = dis2 * (u + y)
        b = t * (v16 + t * la)
        s = dis2 * (vs_c + dis2)
        dn = (((1,), (1,)), ((), ()))
        pre = (a + lax.dot_general(b, we_ref[...], dn,
                                   preferred_element_type=F32)
               + s * be_ref[...])
        out_ref[...] = (lax.dot_general(pre, w_ref[...], dn,
                                        preferred_element_type=F32)
                        + bias_ref[...])

    return pl.pallas_call(
        body,
        grid=grid,
        in_specs=[
            pl.BlockSpec((NC, BLK, 128), lambda i: (0, i, 0)),
            pl.BlockSpec((NC, BLK, 16), lambda i: (0, i, 0)),
            pl.BlockSpec((NW, BLK), lambda i: (0, i)),
            pl.BlockSpec((BLK, 128), lambda i: (i, 0)),
            pl.BlockSpec((BLK, 16), lambda i: (i, 0)),
            pl.BlockSpec((BLK, 32), lambda i: (i, 0)),
            pl.BlockSpec((128, 16), lambda i: (0, 0)),
            pl.BlockSpec((1, 128), lambda i: (0, 0)),
            pl.BlockSpec((128, 128), lambda i: (0, 0)),
            pl.BlockSpec((1, 128), lambda i: (0, 0)),
        ],
        out_specs=pl.BlockSpec((BLK, 128), lambda i: (i, 0)),
        out_shape=jax.ShapeDtypeStruct((n, 128), F32),
    )


def kernel(x, edge_index, edge_attr, W, We, be, bias):
    n, d_in = x.shape
    E = edge_index.shape[1]
    n_pad = -(-(n + 1) // 1024) * 1024

    ei = edge_index.astype(I32)
    attr = edge_attr.astype(F32)
    x32 = x.astype(F32)

    sa, sl, cnt = _make_pass1(E, n, n_pad)(ei, attr)
    y, t, dis2t, aux = _make_tcmid(n, n_pad)(sa, sl, cnt, x32)
    acc128 = _make_pass2a(E, n, n_pad)(ei, y)
    acc16, vs = _make_pass2b(E, n, n_pad)(ei, attr, t, dis2t)
    return _make_tcpost(n, n_pad)(acc128, acc16, vs, y, t, aux,
                                  We.astype(F32), be.astype(F32)[None],
                                  W.astype(F32), bias.astype(F32)[None])
